# baseline (speedup 1.0000x reference)
import functools

import jax
import jax.numpy as jnp
from jax import lax
from jax.experimental import pallas as pl
from jax.experimental.pallas import tpu as pltpu
from jax.experimental.pallas import tpu_sc as plsc

MAX_LEN = 4096
D_MODEL = 1024
BATCH = 4

NUM_CORES = 2
NUM_SUBCORES = 16
NUM_WORKERS = NUM_CORES * NUM_SUBCORES
ROWS_PER_WORKER = MAX_LEN // NUM_WORKERS
CHUNK_ROWS = 64


@functools.partial(jax.jit, static_argnames=())
def _broadcast_table(emb_weight):
    mesh = plsc.VectorSubcoreMesh(core_axis_name="c", subcore_axis_name="s")

    @functools.partial(
        pl.kernel,
        mesh=mesh,
        out_type=jax.ShapeDtypeStruct((BATCH, MAX_LEN, D_MODEL), jnp.float32),
        scratch_types=[pltpu.VMEM((CHUNK_ROWS, D_MODEL), jnp.float32)],
    )
    def k(table_hbm, out_hbm, buf):
        wid = lax.axis_index("s") * NUM_CORES + lax.axis_index("c")
        base = wid * ROWS_PER_WORKER
        for c in range(ROWS_PER_WORKER // CHUNK_ROWS):
            r0 = base + c * CHUNK_ROWS
            pltpu.sync_copy(table_hbm.at[pl.ds(r0, CHUNK_ROWS), :], buf)
            for b in range(BATCH):
                pltpu.sync_copy(buf, out_hbm.at[b, pl.ds(r0, CHUNK_ROWS), :])

    return k(emb_weight)


def kernel(x, emb_weight):
    del x
    return _broadcast_table(emb_weight)
